# baseline (speedup 1.0000x reference)
import functools
import math

import jax
import jax.numpy as jnp
from jax import lax
from jax.experimental import pallas as pl
from jax.experimental.pallas import tpu as pltpu
from jax.experimental.pallas import tpu_sc as plsc

V = 32768
N = 4096
PAD = 0
EPS = 0.1 / (V - 2)
COEF = 1.0 - 0.1 - EPS
CONST = 0.9 * math.log(0.9) + 0.1 * math.log(EPS)

R = 256
KBLK = 8192
NR = N // R
NK = V // KBLK


def _tc_body(yhat_ref, tgt_ref, out_ref, acc_ref):
    r = pl.program_id(0)
    k = pl.program_id(1)
    full = jnp.sum(yhat_ref[...], axis=1, keepdims=True)

    @pl.when(k == 0)
    def _():
        acc_ref[...] = full - yhat_ref[:, 0:1]

    @pl.when(k > 0)
    def _():
        acc_ref[...] += full

    @pl.when(k == NK - 1)
    def _():
        mask = tgt_ref[0] != PAD
        total = jnp.sum(
            jnp.where(mask, CONST - EPS * acc_ref[...], 0.0)).reshape(1, 1)

        @pl.when(r == 0)
        def _():
            out_ref[...] = total

        @pl.when(r > 0)
        def _():
            out_ref[...] += total


def _tc_dense_part(yhat, tgt3):
    out = pl.pallas_call(
        _tc_body,
        grid=(NR, NK),
        in_specs=[
            pl.BlockSpec((R, KBLK), lambda r, k: (r, k)),
            pl.BlockSpec((1, R, 1), lambda r, k: (r, 0, 0)),
        ],
        out_specs=pl.BlockSpec((1, 1), lambda r, k: (0, 0)),
        out_shape=jax.ShapeDtypeStruct((1, 1), jnp.float32),
        scratch_shapes=[pltpu.VMEM((R, 1), jnp.float32)],
        compiler_params=pltpu.CompilerParams(
            dimension_semantics=("arbitrary", "arbitrary")),
    )(yhat, tgt3)
    return out[0, 0]


_INFO = plsc.get_sparse_core_info()
NC, NS, L = _INFO.num_cores, _INFO.num_subcores, _INFO.num_lanes
NW = NC * NS
BPW = N // NW

_SC_MESH = plsc.VectorSubcoreMesh(core_axis_name="c", subcore_axis_name="s")


@functools.partial(
    pl.kernel,
    mesh=_SC_MESH,
    out_type=jax.ShapeDtypeStruct((NW, L), jnp.float32),
    scratch_types=[
        pltpu.VMEM((BPW,), jnp.int32),
        pltpu.VMEM((BPW,), jnp.int32),
        pltpu.VMEM((BPW,), jnp.float32),
        pltpu.VMEM((L,), jnp.float32),
        pltpu.SemaphoreType.DMA,
    ],
)
def _sc_pick_partials(yhat_flat_hbm, tgt_hbm, out_hbm,
                      tgt_v, idx_v, vals_v, out_v, sem):
    wid = lax.axis_index("s") * NC + lax.axis_index("c")
    base = wid * BPW
    pltpu.sync_copy(tgt_hbm.at[pl.ds(base, BPW)], tgt_v)
    lane = lax.iota(jnp.int32, L)
    for j in range(BPW // L):
        t = tgt_v[pl.ds(j * L, L)]
        row = (base + j * L) + lane
        idx_v[pl.ds(j * L, L)] = row * V + t
    pltpu.async_copy(yhat_flat_hbm.at[idx_v], vals_v, sem).wait()
    acc = jnp.zeros((L,), jnp.float32)
    for j in range(BPW // L):
        t = tgt_v[pl.ds(j * L, L)]
        v = vals_v[pl.ds(j * L, L)]
        acc = acc + jnp.where(t != PAD, v, 0.0)
    out_v[...] = acc
    pltpu.sync_copy(out_v, out_hbm.at[wid])


def kernel(yhat, target):
    pick_partials = _sc_pick_partials(yhat.reshape(-1), target)
    dense = _tc_dense_part(yhat, target.reshape(NR, R, 1))
    return dense - COEF * jnp.sum(pick_partials)


# speedup vs baseline: 1.6403x; 1.6403x over previous
import functools
import math

import jax
import jax.numpy as jnp
from jax import lax
from jax.experimental import pallas as pl
from jax.experimental.pallas import tpu as pltpu
from jax.experimental.pallas import tpu_sc as plsc

V = 32768
N = 4096
PAD = 0
EPS = 0.1 / (V - 2)
COEF = 1.0 - 0.1 - EPS
CONST = 0.9 * math.log(0.9) + 0.1 * math.log(EPS)

_INFO = plsc.get_sparse_core_info()
NC, NS, L = _INFO.num_cores, _INFO.num_subcores, _INFO.num_lanes
NW = NC * NS
RPW = N // NW
SLAB = 8
NSLAB = RPW // SLAB
CHUNK = 4096
NCHUNK = V // CHUNK
VPC = CHUNK // L
UNROLL = 8
BUFW = SLAB * CHUNK

_SC_MESH = plsc.VectorSubcoreMesh(core_axis_name="c", subcore_axis_name="s")


@functools.partial(
    pl.kernel,
    mesh=_SC_MESH,
    out_type=jax.ShapeDtypeStruct((NW, L), jnp.float32),
    scratch_types=[
        pltpu.VMEM((RPW,), jnp.int32),
        pltpu.VMEM((BUFW,), jnp.float32),
        pltpu.VMEM((BUFW,), jnp.float32),
        pltpu.VMEM((2 * L,), jnp.float32),
        pltpu.VMEM((2 * L,), jnp.float32),
        pltpu.VMEM((L,), jnp.float32),
        pltpu.SemaphoreType.DMA,
    ],
)
def _sc_loss(yhat_hbm, tgt_hbm, out_hbm,
             tgt_v, buf0, buf1, sbuf, cbuf, out_v, sem):
    wid = lax.axis_index("s") * NC + lax.axis_index("c")
    row_base = wid * RPW
    pltpu.sync_copy(tgt_hbm.at[pl.ds(row_base, RPW)], tgt_v)
    lane = lax.iota(jnp.int32, L)
    bufs = (buf0, buf1)
    zeros = jnp.zeros((L,), jnp.float32)
    sbuf[pl.ds(L, L)] = zeros

    def start_dma(s, c, buf):
        r0 = row_base + s * SLAB
        for j in range(SLAB):
            pltpu.async_copy(
                yhat_hbm.at[r0 + j, pl.ds(c * CHUNK, CHUNK)],
                buf.at[pl.ds(j * CHUNK, CHUNK)], sem)

    start_dma(0, 0, buf0)

    def slab_body(s, carry):
        accd, accp, cnt = carry
        tvec = tgt_v[pl.ds(s * SLAB, L)]
        live = (lane < SLAB) & (tvec != PAD)
        tsc = [tvec[j] for j in range(SLAB)]
        rowacc = [zeros for _ in range(SLAB)]
        accp_s = zeros
        for c in range(NCHUNK):
            buf = bufs[c % 2]
            pltpu.make_async_copy(
                yhat_hbm.at[row_base, pl.ds(0, BUFW)], buf, sem).wait()
            if c < NCHUNK - 1:
                start_dma(s, c + 1, bufs[(c + 1) % 2])
            else:
                start_dma(jnp.minimum(s + 1, NSLAB - 1), 0, bufs[0])
            for j in range(SLAB):
                def vec_body(i, a, j=j, buf=buf):
                    for q in range(UNROLL):
                        a = a + buf[pl.ds(
                            j * CHUNK + i * (L * UNROLL) + q * L, L)]
                    return a
                rowacc[j] = lax.fori_loop(
                    0, VPC // UNROLL, vec_body, rowacc[j], unroll=1)
                if c == 0:
                    rowacc[j] = rowacc[j] - jnp.where(
                        lane == 0, buf[pl.ds(j * CHUNK, L)], 0.0)
                tadj = jnp.where(tsc[j] == PAD, -1, tsc[j])
                off = jnp.clip(tsc[j] - c * CHUNK, 0, CHUNK - L)
                off = (off // L) * L
                w = buf[pl.ds(j * CHUNK + off, L)]
                hit = (lane + (c * CHUNK + off - tadj)) == 0
                accp_s = accp_s + jnp.where(hit, w, 0.0)
        for j in range(SLAB):
            v = rowacc[j]
            for sh in (8, 4, 2, 1):
                sbuf[pl.ds(0, L)] = v
                v = v + sbuf[pl.ds(sh, L)]
            cbuf[pl.ds(j, L)] = v
        rowsums = cbuf[pl.ds(0, L)]
        accd = accd + jnp.where(live, rowsums, 0.0)
        cnt = cnt + jnp.where(live, 1.0, 0.0)
        return accd, accp + accp_s, cnt

    accd, accp, cnt = lax.fori_loop(
        0, NSLAB, slab_body, (zeros, zeros, zeros), unroll=1)
    pltpu.make_async_copy(
        yhat_hbm.at[row_base, pl.ds(0, BUFW)], buf0, sem).wait()

    out_v[...] = CONST * cnt - COEF * accp - EPS * accd
    pltpu.sync_copy(out_v, out_hbm.at[wid])


def kernel(yhat, target):
    partials = _sc_loss(yhat, target)
    return jnp.sum(partials)


# speedup vs baseline: 2.7228x; 1.6599x over previous
import functools
import math

import jax
import jax.numpy as jnp
from jax import lax
from jax.experimental import pallas as pl
from jax.experimental.pallas import tpu as pltpu
from jax.experimental.pallas import tpu_sc as plsc

V = 32768
N = 4096
PAD = 0
EPS = 0.1 / (V - 2)
COEF = 1.0 - 0.1 - EPS
CONST = 0.9 * math.log(0.9) + 0.1 * math.log(EPS)

NT = 2048

R = 256
KBLK = 8192
NR = NT // R
NK = V // KBLK


def _tc_body(yhat_ref, tgt_ref, out_ref, acc_ref):
    r = pl.program_id(0)
    k = pl.program_id(1)
    blk = yhat_ref[...]
    tgt = tgt_ref[0]
    colid = k * KBLK + lax.broadcasted_iota(jnp.int32, blk.shape, 1)
    full = jnp.sum(blk, axis=1, keepdims=True)
    pick = jnp.sum(jnp.where(colid == tgt, blk, 0.0), axis=1, keepdims=True)
    q = EPS * full + COEF * pick

    @pl.when(k == 0)
    def _():
        acc_ref[...] = q - EPS * yhat_ref[:, 0:1]

    @pl.when(k > 0)
    def _():
        acc_ref[...] += q

    @pl.when(k == NK - 1)
    def _():
        mask = tgt != PAD
        total = jnp.sum(jnp.where(mask, CONST - acc_ref[...], 0.0))
        total = total.reshape(1, 1)

        @pl.when(r == 0)
        def _():
            out_ref[...] = total

        @pl.when(r > 0)
        def _():
            out_ref[...] += total


def _tc_part(yhat, tgt3):
    out = pl.pallas_call(
        _tc_body,
        grid=(NR, NK),
        in_specs=[
            pl.BlockSpec((R, KBLK), lambda r, k: (r, k)),
            pl.BlockSpec((1, R, 1), lambda r, k: (r, 0, 0)),
        ],
        out_specs=pl.BlockSpec((1, 1), lambda r, k: (0, 0)),
        out_shape=jax.ShapeDtypeStruct((1, 1), jnp.float32),
        scratch_shapes=[pltpu.VMEM((R, 1), jnp.float32)],
        compiler_params=pltpu.CompilerParams(
            dimension_semantics=("arbitrary", "arbitrary")),
    )(yhat, tgt3)
    return out[0, 0]


_INFO = plsc.get_sparse_core_info()
NC, NS, L = _INFO.num_cores, _INFO.num_subcores, _INFO.num_lanes
NW = NC * NS
RPW = (N - NT) // NW
SLAB = 8
NSLAB = RPW // SLAB
CHUNK = 4096
NCHUNK = V // CHUNK
VPC = CHUNK // L
UNROLL = 8
BUFW = SLAB * CHUNK

_SC_MESH = plsc.VectorSubcoreMesh(core_axis_name="c", subcore_axis_name="s")


@functools.partial(
    pl.kernel,
    mesh=_SC_MESH,
    out_type=jax.ShapeDtypeStruct((NW, L), jnp.float32),
    scratch_types=[
        pltpu.VMEM((RPW,), jnp.int32),
        pltpu.VMEM((BUFW,), jnp.float32),
        pltpu.VMEM((BUFW,), jnp.float32),
        pltpu.VMEM((2 * L,), jnp.float32),
        pltpu.VMEM((2 * L,), jnp.float32),
        pltpu.VMEM((L,), jnp.float32),
        pltpu.SemaphoreType.DMA,
    ],
)
def _sc_loss(yhat_hbm, tgt_hbm, out_hbm,
             tgt_v, buf0, buf1, sbuf, cbuf, out_v, sem):
    wid = lax.axis_index("s") * NC + lax.axis_index("c")
    row_base = NT + wid * RPW
    pltpu.sync_copy(tgt_hbm.at[pl.ds(row_base, RPW)], tgt_v)
    lane = lax.iota(jnp.int32, L)
    bufs = (buf0, buf1)
    zeros = jnp.zeros((L,), jnp.float32)
    sbuf[pl.ds(L, L)] = zeros

    def start_dma(s, c, buf):
        r0 = row_base + s * SLAB
        for j in range(SLAB):
            pltpu.async_copy(
                yhat_hbm.at[r0 + j, pl.ds(c * CHUNK, CHUNK)],
                buf.at[pl.ds(j * CHUNK, CHUNK)], sem)

    start_dma(0, 0, buf0)

    def slab_body(s, carry):
        accd, accp, cnt = carry
        tvec = tgt_v[pl.ds(s * SLAB, L)]
        live = (lane < SLAB) & (tvec != PAD)
        tsc = [tvec[j] for j in range(SLAB)]
        rowacc = [zeros for _ in range(SLAB)]
        accp_s = zeros
        for c in range(NCHUNK):
            buf = bufs[c % 2]
            pltpu.make_async_copy(
                yhat_hbm.at[row_base, pl.ds(0, BUFW)], buf, sem).wait()
            if c < NCHUNK - 1:
                start_dma(s, c + 1, bufs[(c + 1) % 2])
            else:
                start_dma(jnp.minimum(s + 1, NSLAB - 1), 0, bufs[0])
            for j in range(SLAB):
                def vec_body(i, a, j=j, buf=buf):
                    for q in range(UNROLL):
                        a = a + buf[pl.ds(
                            j * CHUNK + i * (L * UNROLL) + q * L, L)]
                    return a
                rowacc[j] = lax.fori_loop(
                    0, VPC // UNROLL, vec_body, rowacc[j], unroll=1)
                if c == 0:
                    rowacc[j] = rowacc[j] - jnp.where(
                        lane == 0, buf[pl.ds(j * CHUNK, L)], 0.0)
                tadj = jnp.where(tsc[j] == PAD, -1, tsc[j])
                off = jnp.clip(tsc[j] - c * CHUNK, 0, CHUNK - L)
                off = (off // L) * L
                w = buf[pl.ds(j * CHUNK + off, L)]
                hit = (lane + (c * CHUNK + off - tadj)) == 0
                accp_s = accp_s + jnp.where(hit, w, 0.0)
        for j in range(SLAB):
            v = rowacc[j]
            for sh in (8, 4, 2, 1):
                sbuf[pl.ds(0, L)] = v
                v = v + sbuf[pl.ds(sh, L)]
            cbuf[pl.ds(j, L)] = v
        rowsums = cbuf[pl.ds(0, L)]
        accd = accd + jnp.where(live, rowsums, 0.0)
        cnt = cnt + jnp.where(live, 1.0, 0.0)
        return accd, accp + accp_s, cnt

    accd, accp, cnt = lax.fori_loop(
        0, NSLAB, slab_body, (zeros, zeros, zeros), unroll=1)
    pltpu.make_async_copy(
        yhat_hbm.at[row_base, pl.ds(0, BUFW)], buf0, sem).wait()

    out_v[...] = CONST * cnt - COEF * accp - EPS * accd
    pltpu.sync_copy(out_v, out_hbm.at[wid])


def kernel(yhat, target):
    sc_partials = _sc_loss(yhat, target)
    tc_part = _tc_part(yhat, target[:NT].reshape(NR, R, 1))
    return tc_part + jnp.sum(sc_partials)


# speedup vs baseline: 2.9543x; 1.0851x over previous
import functools
import math

import jax
import jax.numpy as jnp
from jax import lax
from jax.experimental import pallas as pl
from jax.experimental.pallas import tpu as pltpu
from jax.experimental.pallas import tpu_sc as plsc

V = 32768
N = 4096
PAD = 0
EPS = 0.1 / (V - 2)
COEF = 1.0 - 0.1 - EPS
CONST = 0.9 * math.log(0.9) + 0.1 * math.log(EPS)

NT = 3072

R = 256
KBLK = 8192
NR = NT // R
NK = V // KBLK


def _tc_body(yhat_ref, tgt_ref, out_ref, acc_ref):
    r = pl.program_id(0)
    k = pl.program_id(1)
    blk = yhat_ref[...]
    tgt = tgt_ref[0]
    colid = k * KBLK + lax.broadcasted_iota(jnp.int32, blk.shape, 1)
    full = jnp.sum(blk, axis=1, keepdims=True)
    pick = jnp.sum(jnp.where(colid == tgt, blk, 0.0), axis=1, keepdims=True)
    q = EPS * full + COEF * pick

    @pl.when(k == 0)
    def _():
        acc_ref[...] = q - EPS * yhat_ref[:, 0:1]

    @pl.when(k > 0)
    def _():
        acc_ref[...] += q

    @pl.when(k == NK - 1)
    def _():
        mask = tgt != PAD
        total = jnp.sum(jnp.where(mask, CONST - acc_ref[...], 0.0))
        total = total.reshape(1, 1)

        @pl.when(r == 0)
        def _():
            out_ref[...] = total

        @pl.when(r > 0)
        def _():
            out_ref[...] += total


def _tc_part(yhat, tgt3):
    out = pl.pallas_call(
        _tc_body,
        grid=(NR, NK),
        in_specs=[
            pl.BlockSpec((R, KBLK), lambda r, k: (r, k)),
            pl.BlockSpec((1, R, 1), lambda r, k: (r, 0, 0)),
        ],
        out_specs=pl.BlockSpec((1, 1), lambda r, k: (0, 0)),
        out_shape=jax.ShapeDtypeStruct((1, 1), jnp.float32),
        scratch_shapes=[pltpu.VMEM((R, 1), jnp.float32)],
        compiler_params=pltpu.CompilerParams(
            dimension_semantics=("arbitrary", "arbitrary")),
    )(yhat, tgt3)
    return out[0, 0]


_INFO = plsc.get_sparse_core_info()
NC, NS, L = _INFO.num_cores, _INFO.num_subcores, _INFO.num_lanes
NW = NC * NS
RPW = (N - NT) // NW
SLAB = 8
NSLAB = RPW // SLAB
CHUNK = 4096
NCHUNK = V // CHUNK
VPC = CHUNK // L
UNROLL = 8
BUFW = SLAB * CHUNK

_SC_MESH = plsc.VectorSubcoreMesh(core_axis_name="c", subcore_axis_name="s")


@functools.partial(
    pl.kernel,
    mesh=_SC_MESH,
    out_type=jax.ShapeDtypeStruct((NW, L), jnp.float32),
    scratch_types=[
        pltpu.VMEM((RPW,), jnp.int32),
        pltpu.VMEM((BUFW,), jnp.float32),
        pltpu.VMEM((BUFW,), jnp.float32),
        pltpu.VMEM((2 * L,), jnp.float32),
        pltpu.VMEM((2 * L,), jnp.float32),
        pltpu.VMEM((L,), jnp.float32),
        pltpu.SemaphoreType.DMA,
    ],
)
def _sc_loss(yhat_hbm, tgt_hbm, out_hbm,
             tgt_v, buf0, buf1, sbuf, cbuf, out_v, sem):
    wid = lax.axis_index("s") * NC + lax.axis_index("c")
    row_base = NT + wid * RPW
    pltpu.sync_copy(tgt_hbm.at[pl.ds(row_base, RPW)], tgt_v)
    lane = lax.iota(jnp.int32, L)
    bufs = (buf0, buf1)
    zeros = jnp.zeros((L,), jnp.float32)
    sbuf[pl.ds(L, L)] = zeros

    def start_dma(s, c, buf):
        r0 = row_base + s * SLAB
        for j in range(SLAB):
            pltpu.async_copy(
                yhat_hbm.at[r0 + j, pl.ds(c * CHUNK, CHUNK)],
                buf.at[pl.ds(j * CHUNK, CHUNK)], sem)

    start_dma(0, 0, buf0)

    def slab_body(s, carry):
        accd, accp, cnt = carry
        tvec = tgt_v[pl.ds(s * SLAB, L)]
        live = (lane < SLAB) & (tvec != PAD)
        tsc = [tvec[j] for j in range(SLAB)]
        rowacc = [zeros for _ in range(SLAB)]
        accp_s = zeros
        for c in range(NCHUNK):
            buf = bufs[c % 2]
            pltpu.make_async_copy(
                yhat_hbm.at[row_base, pl.ds(0, BUFW)], buf, sem).wait()
            if c < NCHUNK - 1:
                start_dma(s, c + 1, bufs[(c + 1) % 2])
            else:
                start_dma(jnp.minimum(s + 1, NSLAB - 1), 0, bufs[0])
            for j in range(SLAB):
                def vec_body(i, a, j=j, buf=buf):
                    for q in range(UNROLL):
                        a = a + buf[pl.ds(
                            j * CHUNK + i * (L * UNROLL) + q * L, L)]
                    return a
                rowacc[j] = lax.fori_loop(
                    0, VPC // UNROLL, vec_body, rowacc[j], unroll=1)
                if c == 0:
                    rowacc[j] = rowacc[j] - jnp.where(
                        lane == 0, buf[pl.ds(j * CHUNK, L)], 0.0)
                tadj = jnp.where(tsc[j] == PAD, -1, tsc[j])
                off = jnp.clip(tsc[j] - c * CHUNK, 0, CHUNK - L)
                off = (off // L) * L
                w = buf[pl.ds(j * CHUNK + off, L)]
                hit = (lane + (c * CHUNK + off - tadj)) == 0
                accp_s = accp_s + jnp.where(hit, w, 0.0)
        for j in range(SLAB):
            v = rowacc[j]
            for sh in (8, 4, 2, 1):
                sbuf[pl.ds(0, L)] = v
                v = v + sbuf[pl.ds(sh, L)]
            cbuf[pl.ds(j, L)] = v
        rowsums = cbuf[pl.ds(0, L)]
        accd = accd + jnp.where(live, rowsums, 0.0)
        cnt = cnt + jnp.where(live, 1.0, 0.0)
        return accd, accp + accp_s, cnt

    accd, accp, cnt = lax.fori_loop(
        0, NSLAB, slab_body, (zeros, zeros, zeros), unroll=1)
    pltpu.make_async_copy(
        yhat_hbm.at[row_base, pl.ds(0, BUFW)], buf0, sem).wait()

    out_v[...] = CONST * cnt - COEF * accp - EPS * accd
    pltpu.sync_copy(out_v, out_hbm.at[wid])


def kernel(yhat, target):
    sc_partials = _sc_loss(yhat, target)
    tc_part = _tc_part(yhat, target[:NT].reshape(NR, R, 1))
    return tc_part + jnp.sum(sc_partials)


# speedup vs baseline: 3.0586x; 1.0353x over previous
import functools
import math

import jax
import jax.numpy as jnp
from jax import lax
from jax.experimental import pallas as pl
from jax.experimental.pallas import tpu as pltpu
from jax.experimental.pallas import tpu_sc as plsc

V = 32768
N = 4096
PAD = 0
EPS = 0.1 / (V - 2)
COEF = 1.0 - 0.1 - EPS
CONST = 0.9 * math.log(0.9) + 0.1 * math.log(EPS)

R = 256
KBLK = 8192
NR = N // R
NK = V // KBLK


def _tc_body(yhat_ref, tgt_ref, out_ref, acc_ref):
    r = pl.program_id(0)
    k = pl.program_id(1)
    full = jnp.sum(yhat_ref[...], axis=1, keepdims=True)

    @pl.when(k == 0)
    def _():
        acc_ref[...] = full - yhat_ref[:, 0:1]

    @pl.when(k > 0)
    def _():
        acc_ref[...] += full

    @pl.when(k == NK - 1)
    def _():
        mask = tgt_ref[0] != PAD
        total = jnp.sum(
            jnp.where(mask, -EPS * acc_ref[...], 0.0)).reshape(1, 1)

        @pl.when(r == 0)
        def _():
            out_ref[...] = total

        @pl.when(r > 0)
        def _():
            out_ref[...] += total


def _tc_dense(yhat, tgt3):
    out = pl.pallas_call(
        _tc_body,
        grid=(NR, NK),
        in_specs=[
            pl.BlockSpec((R, KBLK), lambda r, k: (r, k)),
            pl.BlockSpec((1, R, 1), lambda r, k: (r, 0, 0)),
        ],
        out_specs=pl.BlockSpec((1, 1), lambda r, k: (0, 0)),
        out_shape=jax.ShapeDtypeStruct((1, 1), jnp.float32),
        scratch_shapes=[pltpu.VMEM((R, 1), jnp.float32)],
        compiler_params=pltpu.CompilerParams(
            dimension_semantics=("arbitrary", "arbitrary")),
    )(yhat, tgt3)
    return out[0, 0]


_INFO = plsc.get_sparse_core_info()
NC, NS, L = _INFO.num_cores, _INFO.num_subcores, _INFO.num_lanes
NW = NC * NS
RPW = N // NW
NG = RPW // L

_SC_MESH = plsc.VectorSubcoreMesh(core_axis_name="c", subcore_axis_name="s")


@functools.partial(
    pl.kernel,
    mesh=_SC_MESH,
    out_type=jax.ShapeDtypeStruct((NW, L), jnp.float32),
    scratch_types=[
        pltpu.VMEM((RPW,), jnp.int32),
        pltpu.VMEM((RPW * L,), jnp.float32),
        pltpu.VMEM((L,), jnp.float32),
        pltpu.SemaphoreType.DMA,
    ],
)
def _sc_pick(yhat_hbm, tgt_hbm, out_hbm, tgt_v, wbuf, out_v, sem):
    wid = lax.axis_index("s") * NC + lax.axis_index("c")
    row_base = wid * RPW
    pltpu.sync_copy(tgt_hbm.at[pl.ds(row_base, RPW)], tgt_v)
    lane = lax.iota(jnp.int32, L)
    accp = jnp.zeros((L,), jnp.float32)
    cnt = jnp.zeros((L,), jnp.float32)
    for g in range(NG):
        tvec = tgt_v[pl.ds(g * L, L)]
        offs = []
        handles = []
        for j in range(L):
            toff = (tvec[j] // L) * L
            offs.append(toff)
            handles.append(pltpu.async_copy(
                yhat_hbm.at[row_base + g * L + j, pl.ds(toff, L)],
                wbuf.at[pl.ds((g * L + j) * L, L)], sem))
        for h in handles:
            h.wait()
        for j in range(L):
            tadj = jnp.where(tvec[j] == PAD, -1, tvec[j])
            w = wbuf[pl.ds((g * L + j) * L, L)]
            hit = (lane + (offs[j] - tadj)) == 0
            accp = accp + jnp.where(hit, w, 0.0)
        cnt = cnt + jnp.where(tvec != PAD, 1.0, 0.0)
    out_v[...] = CONST * cnt - COEF * accp
    pltpu.sync_copy(out_v, out_hbm.at[wid])


def kernel(yhat, target):
    sc_partials = _sc_pick(yhat, target)
    tc_part = _tc_dense(yhat, target.reshape(NR, R, 1))
    return tc_part + jnp.sum(sc_partials)


# speedup vs baseline: 3.0849x; 1.0086x over previous
import functools
import math

import jax
import jax.numpy as jnp
from jax import lax
from jax.experimental import pallas as pl
from jax.experimental.pallas import tpu as pltpu
from jax.experimental.pallas import tpu_sc as plsc

V = 32768
N = 4096
PAD = 0
EPS = 0.1 / (V - 2)
COEF = 1.0 - 0.1 - EPS
CONST = 0.9 * math.log(0.9) + 0.1 * math.log(EPS)

R = 128
NR = N // R


def _tc_body(yhat_ref, tgt_ref, out_ref):
    r = pl.program_id(0)
    acc = jnp.sum(yhat_ref[...], axis=1, keepdims=True) - yhat_ref[:, 0:1]
    mask = tgt_ref[0] != PAD
    total = jnp.sum(jnp.where(mask, -EPS * acc, 0.0)).reshape(1, 1)

    @pl.when(r == 0)
    def _():
        out_ref[...] = total

    @pl.when(r > 0)
    def _():
        out_ref[...] += total


def _tc_dense(yhat, tgt3):
    out = pl.pallas_call(
        _tc_body,
        grid=(NR,),
        in_specs=[
            pl.BlockSpec((R, V), lambda r: (r, 0)),
            pl.BlockSpec((1, R, 1), lambda r: (r, 0, 0)),
        ],
        out_specs=pl.BlockSpec((1, 1), lambda r: (0, 0)),
        out_shape=jax.ShapeDtypeStruct((1, 1), jnp.float32),
        compiler_params=pltpu.CompilerParams(
            dimension_semantics=("arbitrary",)),
    )(yhat, tgt3)
    return out[0, 0]


_INFO = plsc.get_sparse_core_info()
NC, NS, L = _INFO.num_cores, _INFO.num_subcores, _INFO.num_lanes
NW = NC * NS
RPW = N // NW
NG = RPW // L

_SC_MESH = plsc.VectorSubcoreMesh(core_axis_name="c", subcore_axis_name="s")


@functools.partial(
    pl.kernel,
    mesh=_SC_MESH,
    out_type=jax.ShapeDtypeStruct((NW, L), jnp.float32),
    scratch_types=[
        pltpu.VMEM((RPW,), jnp.int32),
        pltpu.VMEM((RPW * L,), jnp.float32),
        pltpu.VMEM((L,), jnp.float32),
        pltpu.SemaphoreType.DMA,
    ],
)
def _sc_pick(yhat_hbm, tgt_hbm, out_hbm, tgt_v, wbuf, out_v, sem):
    wid = lax.axis_index("s") * NC + lax.axis_index("c")
    row_base = wid * RPW
    pltpu.sync_copy(tgt_hbm.at[pl.ds(row_base, RPW)], tgt_v)
    lane = lax.iota(jnp.int32, L)
    accp = jnp.zeros((L,), jnp.float32)
    cnt = jnp.zeros((L,), jnp.float32)
    for g in range(NG):
        tvec = tgt_v[pl.ds(g * L, L)]
        offs = []
        handles = []
        for j in range(L):
            toff = (tvec[j] // L) * L
            offs.append(toff)
            handles.append(pltpu.async_copy(
                yhat_hbm.at[row_base + g * L + j, pl.ds(toff, L)],
                wbuf.at[pl.ds((g * L + j) * L, L)], sem))
        for h in handles:
            h.wait()
        for j in range(L):
            tadj = jnp.where(tvec[j] == PAD, -1, tvec[j])
            w = wbuf[pl.ds((g * L + j) * L, L)]
            hit = (lane + (offs[j] - tadj)) == 0
            accp = accp + jnp.where(hit, w, 0.0)
        cnt = cnt + jnp.where(tvec != PAD, 1.0, 0.0)
    out_v[...] = CONST * cnt - COEF * accp
    pltpu.sync_copy(out_v, out_hbm.at[wid])


def kernel(yhat, target):
    sc_partials = _sc_pick(yhat, target)
    tc_part = _tc_dense(yhat, target.reshape(NR, R, 1))
    return tc_part + jnp.sum(sc_partials)
